# baseline (speedup 1.0000x reference)
import functools

import numpy as np

import jax
import jax.numpy as jnp
from jax import lax
from jax.experimental import pallas as pl
from jax.experimental.pallas import tpu as pltpu
from jax.experimental.pallas import tpu_sc as plsc

_LANES = 16


def _matmul_body(x_ref, w_ref, o_ref):
    o_ref[...] = jnp.dot(x_ref[...], w_ref[...],
                         preferred_element_type=jnp.float32
                         ).astype(o_ref.dtype)


def _tc_matmul(x, w, bn, out_dtype):
    n, k = x.shape
    kk, m = w.shape
    return pl.pallas_call(
        _matmul_body,
        grid=(n // bn,),
        in_specs=[
            pl.BlockSpec((bn, k), lambda i: (i, 0)),
            pl.BlockSpec((kk, m), lambda i: (0, 0)),
        ],
        out_specs=pl.BlockSpec((bn, m), lambda i: (i, 0)),
        out_shape=jax.ShapeDtypeStruct((n, m), out_dtype),
    )(x, w)


def _gru_body(c, agg_ref, h_ref, wih_ref, whh_ref, bih_ref, bhh_ref, o_ref):
    h = h_ref[...]
    gi = jnp.dot(agg_ref[...], wih_ref[...],
                 preferred_element_type=jnp.float32) + bih_ref[...]
    gh = jnp.dot(h, whh_ref[...],
                 preferred_element_type=jnp.float32) + bhh_ref[...]
    r = jax.nn.sigmoid(gi[:, :c] + gh[:, :c])
    z = jax.nn.sigmoid(gi[:, c:2 * c] + gh[:, c:2 * c])
    nn = jnp.tanh(gi[:, 2 * c:] + r * gh[:, 2 * c:])
    o_ref[...] = (1.0 - z) * nn + z * h


def _tc_gru(agg, h, wih_t, whh_t, bih, bhh, bn):
    n, c = h.shape
    g3 = wih_t.shape[1]
    return pl.pallas_call(
        functools.partial(_gru_body, c),
        grid=(n // bn,),
        in_specs=[
            pl.BlockSpec((bn, c), lambda i: (i, 0)),
            pl.BlockSpec((bn, c), lambda i: (i, 0)),
            pl.BlockSpec((c, g3), lambda i: (0, 0)),
            pl.BlockSpec((c, g3), lambda i: (0, 0)),
            pl.BlockSpec((1, g3), lambda i: (0, 0)),
            pl.BlockSpec((1, g3), lambda i: (0, 0)),
        ],
        out_specs=pl.BlockSpec((bn, c), lambda i: (i, 0)),
        out_shape=jax.ShapeDtypeStruct((n, c), jnp.float32),
    )(agg, h, wih_t, whh_t, bih, bhh)


def _sc_gather_sum(mw, idx2, deg, nc, ns, dpw0):
    npad2, cw = mw.shape
    npad = npad2 // 2
    c = 2 * cw
    rows_per_step = idx2.shape[1]
    sub = rows_per_step // deg
    dpw1 = npad // ns - dpw0
    steps0 = dpw0 // sub
    steps1 = dpw1 // sub
    steps_max = max(steps0, steps1)
    dpw_max = max(dpw0, dpw1)
    core0_total = ns * dpw0
    wgroups = cw // _LANES
    mesh = plsc.VectorSubcoreMesh(core_axis_name="c", subcore_axis_name="s")

    @functools.partial(
        pl.kernel,
        out_type=jax.ShapeDtypeStruct((npad, c), jnp.float32),
        mesh=mesh,
        compiler_params=pltpu.CompilerParams(use_tc_tiling_on_sc=False),
        scratch_types=[
            pltpu.VMEM((steps_max, rows_per_step), jnp.int32),
            pltpu.VMEM((4, rows_per_step, cw), jnp.int32),
            pltpu.VMEM((dpw_max, c), jnp.float32),
            pltpu.SemaphoreType.DMA,
            pltpu.SemaphoreType.DMA,
            pltpu.SemaphoreType.DMA,
            pltpu.SemaphoreType.DMA,
        ],
    )
    def k(m_hbm, idx_hbm, out_hbm, idx_v, rows_v, out_v,
          sem0, sem1, sem2, sem3):
        sid = lax.axis_index("s")
        cid = lax.axis_index("c")
        on0 = cid == 0
        dst0 = jnp.where(on0, sid * dpw0, core0_total + sid * dpw1)
        mysteps = jnp.where(on0, steps0, steps1)
        row0 = dst0 // sub

        if steps0 > 0:
            @pl.when(on0)
            def _():
                pltpu.sync_copy(idx_hbm.at[pl.ds(row0, steps0)],
                                idx_v.at[pl.ds(0, steps0)])

        if steps1 > 0:
            @pl.when(jnp.logical_not(on0))
            def _():
                pltpu.sync_copy(idx_hbm.at[pl.ds(row0, steps1)],
                                idx_v.at[pl.ds(0, steps1)])
                shift = jnp.full((_LANES,), npad, jnp.int32)

                def sbody(g2, carry2):
                    for v in range(rows_per_step // _LANES):
                        sl = pl.ds(v * _LANES, _LANES)
                        idx_v[g2, sl] = idx_v[g2, sl] + shift
                    return carry2

                lax.fori_loop(0, steps1, sbody, 0)

        sems = (sem0, sem1, sem2, sem3)
        nbuf = len(sems)
        himask = jnp.full((_LANES,), -65536, jnp.int32)

        def start(g, b):
            pltpu.async_copy(m_hbm.at[idx_v.at[g]], rows_v.at[b], sems[b])

        def wait(g, b):
            pltpu.make_async_copy(m_hbm.at[idx_v.at[g]], rows_v.at[b],
                                  sems[b]).wait()

        @pl.when(mysteps >= nbuf)
        def _():
            for b in range(nbuf):
                start(b, b)

        def body(i, carry):
            for b in range(nbuf):
                g = i * nbuf + b
                wait(g, b)
                unroll = 8
                for d in range(sub):
                    def nbody(jo, acc):
                        r0 = d * deg + jo * unroll
                        for u in range(unroll):
                            for v in range(wgroups):
                                w = rows_v[b, r0 + u,
                                           pl.ds(v * _LANES, _LANES)]
                                lo = lax.bitcast_convert_type(
                                    w << 16, jnp.float32)
                                hi = lax.bitcast_convert_type(
                                    w & himask, jnp.float32)
                                acc = (acc[:2 * v]
                                       + (acc[2 * v] + lo,
                                          acc[2 * v + 1] + hi)
                                       + acc[2 * v + 2:])
                        return acc
                    acc = lax.fori_loop(
                        0, deg // unroll, nbody,
                        tuple(jnp.zeros((_LANES,), jnp.float32)
                              for _ in range(2 * wgroups)))
                    row_out = g * sub + d
                    for v in range(wgroups):
                        out_v[row_out,
                              pl.ds(v * 32, _LANES)] = acc[2 * v]
                        out_v[row_out,
                              pl.ds(v * 32 + _LANES, _LANES)] = acc[2 * v + 1]

                @pl.when(g + nbuf < mysteps)
                def _():
                    start(g + nbuf, b)
            return carry

        lax.fori_loop(0, mysteps // nbuf, body, 0)

        if dpw0 > 0:
            @pl.when(on0)
            def _():
                pltpu.sync_copy(out_v.at[pl.ds(0, dpw0)],
                                out_hbm.at[pl.ds(dst0, dpw0)])

        if dpw1 > 0:
            @pl.when(jnp.logical_not(on0))
            def _():
                pltpu.sync_copy(out_v.at[pl.ds(0, dpw1)],
                                out_hbm.at[pl.ds(dst0, dpw1)])

    return k(mw, idx2)


def _perm(c):
    p = []
    for v in range(c // 32):
        p.extend(v * 32 + 2 * k for k in range(16))
        p.extend(v * 32 + 2 * k + 1 for k in range(16))
    return np.asarray(p, np.int32)


def kernel(x, edge_index, weight, W_ih, W_hh, b_ih, b_hh):
    n, c = x.shape
    deg = edge_index.shape[1]
    num_layers = weight.shape[0]
    info = plsc.get_sparse_core_info()
    nc, ns = info.num_cores, info.num_subcores
    nw = nc * ns

    rows_per_step = 128
    sub = rows_per_step // deg
    per_w = sub * nw
    steps = -(-n // per_w)
    steps = -(-steps // 4) * 4
    npad = steps * per_w

    xp = jnp.concatenate(
        [x, jnp.zeros((npad - n, c), jnp.float32)], axis=0)
    ei = jnp.concatenate(
        [edge_index, jnp.zeros((npad - n, deg), jnp.int32)], axis=0)
    idx2 = ei.reshape(npad // sub, rows_per_step)
    dpw0 = (npad // ns) // 2 // 16 * 16

    wih_t = W_ih.T[_perm(c)]
    whh_t = W_hh.T
    bih = b_ih.reshape(1, -1)
    bhh = b_hh.reshape(1, -1)

    bn = 256
    for i in range(num_layers):
        m = _tc_matmul(xp, weight[i], bn, jnp.bfloat16)
        mw = lax.bitcast_convert_type(
            m.reshape(npad, c // 2, 2), jnp.int32)
        mwd = jnp.concatenate([mw, mw], axis=0)
        agg = _sc_gather_sum(mwd, idx2, deg, nc, ns, dpw0)
        xp = _tc_gru(agg, xp, wih_t, whh_t, bih, bhh, bn)
    return xp[:n]


# speedup vs baseline: 1.1928x; 1.1928x over previous
import functools

import numpy as np

import jax
import jax.numpy as jnp
from jax import lax
from jax.experimental import pallas as pl
from jax.experimental.pallas import tpu as pltpu
from jax.experimental.pallas import tpu_sc as plsc

_LANES = 16


def _matmul_body(x_ref, w_ref, o_ref):
    o_ref[...] = jnp.dot(x_ref[...], w_ref[...],
                         preferred_element_type=jnp.float32
                         ).astype(o_ref.dtype)


def _tc_matmul(x, w, bn, out_dtype):
    n, k = x.shape
    kk, m = w.shape
    return pl.pallas_call(
        _matmul_body,
        grid=(n // bn,),
        in_specs=[
            pl.BlockSpec((bn, k), lambda i: (i, 0)),
            pl.BlockSpec((kk, m), lambda i: (0, 0)),
        ],
        out_specs=pl.BlockSpec((bn, m), lambda i: (i, 0)),
        out_shape=jax.ShapeDtypeStruct((n, m), out_dtype),
    )(x, w)


def _gru_body(c, agg_ref, h_ref, wih_ref, whh_ref, bih_ref, bhh_ref, *o_refs):
    h = h_ref[...]
    gi = jnp.dot(agg_ref[...], wih_ref[...],
                 preferred_element_type=jnp.float32) + bih_ref[...]
    gh = jnp.dot(h, whh_ref[...],
                 preferred_element_type=jnp.float32) + bhh_ref[...]
    r = jax.nn.sigmoid(gi[:, :c] + gh[:, :c])
    z = jax.nn.sigmoid(gi[:, c:2 * c] + gh[:, c:2 * c])
    nn = jnp.tanh(gi[:, 2 * c:] + r * gh[:, 2 * c:])
    hn = (1.0 - z) * nn + z * h
    o_refs[0][...] = hn
    if len(o_refs) > 1:
        wn_ref = o_refs[2]
        o_refs[1][...] = jnp.dot(
            hn, wn_ref[...],
            preferred_element_type=jnp.float32).astype(jnp.bfloat16)


def _tc_gru(agg, h, wih_t, whh_t, bih, bhh, bn, w_next=None):
    n, c = h.shape
    g3 = wih_t.shape[1]
    in_specs = [
        pl.BlockSpec((bn, c), lambda i: (i, 0)),
        pl.BlockSpec((bn, c), lambda i: (i, 0)),
        pl.BlockSpec((c, g3), lambda i: (0, 0)),
        pl.BlockSpec((c, g3), lambda i: (0, 0)),
        pl.BlockSpec((1, g3), lambda i: (0, 0)),
        pl.BlockSpec((1, g3), lambda i: (0, 0)),
    ]
    out_specs = pl.BlockSpec((bn, c), lambda i: (i, 0))
    out_shape = jax.ShapeDtypeStruct((n, c), jnp.float32)
    args = [agg, h, wih_t, whh_t, bih, bhh]
    if w_next is not None:

        def body(c_, agg_r, h_r, wih_r, whh_r, bih_r, bhh_r, wn_r, o1, o2):
            _gru_body(c_, agg_r, h_r, wih_r, whh_r, bih_r, bhh_r, o1, o2, wn_r)

        return pl.pallas_call(
            functools.partial(body, c),
            grid=(n // bn,),
            in_specs=in_specs + [pl.BlockSpec((c, c), lambda i: (0, 0))],
            out_specs=[out_specs,
                       pl.BlockSpec((bn, c), lambda i: (i, 0))],
            out_shape=[out_shape,
                       jax.ShapeDtypeStruct((n, c), jnp.bfloat16)],
        )(*args, w_next)
    return pl.pallas_call(
        functools.partial(_gru_body, c),
        grid=(n // bn,),
        in_specs=in_specs,
        out_specs=out_specs,
        out_shape=out_shape,
    )(*args)


def _sc_gather_sum(mw, idx2, deg, nc, ns, dpw0):
    npad, cw = mw.shape
    c = 2 * cw
    rows_per_step = idx2.shape[1]
    sub = rows_per_step // deg
    dpw1 = npad // ns - dpw0
    steps0 = dpw0 // sub
    steps1 = dpw1 // sub
    steps_max = max(steps0, steps1)
    dpw_max = max(dpw0, dpw1)
    core0_total = ns * dpw0
    wgroups = cw // _LANES
    mesh = plsc.VectorSubcoreMesh(core_axis_name="c", subcore_axis_name="s")

    @functools.partial(
        pl.kernel,
        out_type=jax.ShapeDtypeStruct((npad, c), jnp.float32),
        mesh=mesh,
        compiler_params=pltpu.CompilerParams(use_tc_tiling_on_sc=False),
        scratch_types=[
            pltpu.VMEM((steps_max, rows_per_step), jnp.int32),
            pltpu.VMEM((4, rows_per_step, cw), jnp.int32),
            pltpu.VMEM((dpw_max, c), jnp.float32),
            pltpu.SemaphoreType.DMA,
            pltpu.SemaphoreType.DMA,
            pltpu.SemaphoreType.DMA,
            pltpu.SemaphoreType.DMA,
        ],
    )
    def k(m_hbm, idx_hbm, out_hbm, idx_v, rows_v, out_v,
          sem0, sem1, sem2, sem3):
        sid = lax.axis_index("s")
        cid = lax.axis_index("c")
        on0 = cid == 0
        dst0 = jnp.where(on0, sid * dpw0, core0_total + sid * dpw1)
        mysteps = jnp.where(on0, steps0, steps1)
        row0 = dst0 // sub

        if steps0 > 0:
            @pl.when(on0)
            def _():
                pltpu.sync_copy(idx_hbm.at[pl.ds(row0, steps0)],
                                idx_v.at[pl.ds(0, steps0)])

        if steps1 > 0:
            @pl.when(jnp.logical_not(on0))
            def _():
                pltpu.sync_copy(idx_hbm.at[pl.ds(row0, steps1)],
                                idx_v.at[pl.ds(0, steps1)])

        sems = (sem0, sem1, sem2, sem3)
        nbuf = len(sems)
        himask = jnp.full((_LANES,), -65536, jnp.int32)

        def start(g, b):
            pltpu.async_copy(m_hbm.at[idx_v.at[g]], rows_v.at[b], sems[b])

        def wait(g, b):
            pltpu.make_async_copy(m_hbm.at[idx_v.at[g]], rows_v.at[b],
                                  sems[b]).wait()

        @pl.when(mysteps >= nbuf)
        def _():
            for b in range(nbuf):
                start(b, b)

        def body(i, carry):
            for b in range(nbuf):
                g = i * nbuf + b
                wait(g, b)
                unroll = 8
                for d in range(sub):
                    def nbody(jo, acc):
                        r0 = d * deg + jo * unroll
                        for u in range(unroll):
                            for v in range(wgroups):
                                w = rows_v[b, r0 + u,
                                           pl.ds(v * _LANES, _LANES)]
                                lo = lax.bitcast_convert_type(
                                    w << 16, jnp.float32)
                                hi = lax.bitcast_convert_type(
                                    w & himask, jnp.float32)
                                acc = (acc[:2 * v]
                                       + (acc[2 * v] + lo,
                                          acc[2 * v + 1] + hi)
                                       + acc[2 * v + 2:])
                        return acc
                    acc = lax.fori_loop(
                        0, deg // unroll, nbody,
                        tuple(jnp.zeros((_LANES,), jnp.float32)
                              for _ in range(2 * wgroups)))
                    row_out = g * sub + d
                    for v in range(wgroups):
                        out_v[row_out,
                              pl.ds(v * 32, _LANES)] = acc[2 * v]
                        out_v[row_out,
                              pl.ds(v * 32 + _LANES, _LANES)] = acc[2 * v + 1]

                @pl.when(g + nbuf < mysteps)
                def _():
                    start(g + nbuf, b)
            return carry

        lax.fori_loop(0, mysteps // nbuf, body, 0)

        if dpw0 > 0:
            @pl.when(on0)
            def _():
                pltpu.sync_copy(out_v.at[pl.ds(0, dpw0)],
                                out_hbm.at[pl.ds(dst0, dpw0)])

        if dpw1 > 0:
            @pl.when(jnp.logical_not(on0))
            def _():
                pltpu.sync_copy(out_v.at[pl.ds(0, dpw1)],
                                out_hbm.at[pl.ds(dst0, dpw1)])

    return k(mw, idx2)


def _perm(c):
    p = []
    for v in range(c // 32):
        p.extend(v * 32 + 2 * k for k in range(16))
        p.extend(v * 32 + 2 * k + 1 for k in range(16))
    return np.asarray(p, np.int32)


def kernel(x, edge_index, weight, W_ih, W_hh, b_ih, b_hh):
    n, c = x.shape
    deg = edge_index.shape[1]
    num_layers = weight.shape[0]
    info = plsc.get_sparse_core_info()
    nc, ns = info.num_cores, info.num_subcores
    nw = nc * ns

    rows_per_step = 128
    sub = rows_per_step // deg
    per_w = sub * nw
    steps = -(-n // per_w)
    steps = -(-steps // 4) * 4
    npad = steps * per_w

    xp = jnp.concatenate(
        [x, jnp.zeros((npad - n, c), jnp.float32)], axis=0)
    ei = jnp.concatenate(
        [edge_index, jnp.zeros((npad - n, deg), jnp.int32)], axis=0)
    idx2 = ei.reshape(npad // sub, rows_per_step)
    dpw0 = (npad // ns) * 9 // 10 // 16 * 16

    wih_t = W_ih.T[_perm(c)]
    whh_t = W_hh.T
    bih = b_ih.reshape(1, -1)
    bhh = b_hh.reshape(1, -1)

    bn = 256
    m = _tc_matmul(xp, weight[0], bn, jnp.bfloat16)
    for i in range(num_layers):
        mw = lax.bitcast_convert_type(
            m.reshape(npad, c // 2, 2), jnp.int32)
        agg = _sc_gather_sum(mw, idx2, deg, nc, ns, dpw0)
        if i + 1 < num_layers:
            xp, m = _tc_gru(agg, xp, wih_t, whh_t, bih, bhh, bn,
                            w_next=weight[i + 1])
        else:
            xp = _tc_gru(agg, xp, wih_t, whh_t, bih, bhh, bn)
    return xp[:n]


# speedup vs baseline: 1.2026x; 1.0082x over previous
import functools

import numpy as np

import jax
import jax.numpy as jnp
from jax import lax
from jax.experimental import pallas as pl
from jax.experimental.pallas import tpu as pltpu
from jax.experimental.pallas import tpu_sc as plsc

_LANES = 16


def _matmul_body(x_ref, w_ref, o_ref):
    o_ref[...] = jnp.dot(x_ref[...], w_ref[...],
                         preferred_element_type=jnp.float32
                         ).astype(o_ref.dtype)


def _tc_matmul(x, w, bn, out_dtype):
    n, k = x.shape
    kk, m = w.shape
    return pl.pallas_call(
        _matmul_body,
        grid=(n // bn,),
        in_specs=[
            pl.BlockSpec((bn, k), lambda i: (i, 0)),
            pl.BlockSpec((kk, m), lambda i: (0, 0)),
        ],
        out_specs=pl.BlockSpec((bn, m), lambda i: (i, 0)),
        out_shape=jax.ShapeDtypeStruct((n, m), out_dtype),
    )(x, w)


def _gru_body(c, agg_ref, h_ref, wih_ref, whh_ref, bih_ref, bhh_ref, *o_refs):
    h = h_ref[...]
    gi = jnp.dot(agg_ref[...], wih_ref[...],
                 preferred_element_type=jnp.float32) + bih_ref[...]
    gh = jnp.dot(h, whh_ref[...],
                 preferred_element_type=jnp.float32) + bhh_ref[...]
    r = jax.nn.sigmoid(gi[:, :c] + gh[:, :c])
    z = jax.nn.sigmoid(gi[:, c:2 * c] + gh[:, c:2 * c])
    nn = jnp.tanh(gi[:, 2 * c:] + r * gh[:, 2 * c:])
    hn = (1.0 - z) * nn + z * h
    o_refs[0][...] = hn
    if len(o_refs) > 1:
        wn_ref = o_refs[2]
        o_refs[1][...] = jnp.dot(
            hn, wn_ref[...],
            preferred_element_type=jnp.float32).astype(jnp.bfloat16)


def _tc_gru(agg, h, wih_t, whh_t, bih, bhh, bn, w_next=None):
    n, c = h.shape
    g3 = wih_t.shape[1]
    in_specs = [
        pl.BlockSpec((bn, c), lambda i: (i, 0)),
        pl.BlockSpec((bn, c), lambda i: (i, 0)),
        pl.BlockSpec((c, g3), lambda i: (0, 0)),
        pl.BlockSpec((c, g3), lambda i: (0, 0)),
        pl.BlockSpec((1, g3), lambda i: (0, 0)),
        pl.BlockSpec((1, g3), lambda i: (0, 0)),
    ]
    out_specs = pl.BlockSpec((bn, c), lambda i: (i, 0))
    out_shape = jax.ShapeDtypeStruct((n, c), jnp.float32)
    args = [agg, h, wih_t, whh_t, bih, bhh]
    if w_next is not None:

        def body(c_, agg_r, h_r, wih_r, whh_r, bih_r, bhh_r, wn_r, o1, o2):
            _gru_body(c_, agg_r, h_r, wih_r, whh_r, bih_r, bhh_r, o1, o2, wn_r)

        return pl.pallas_call(
            functools.partial(body, c),
            grid=(n // bn,),
            in_specs=in_specs + [pl.BlockSpec((c, c), lambda i: (0, 0))],
            out_specs=[out_specs,
                       pl.BlockSpec((bn, c), lambda i: (i, 0))],
            out_shape=[out_shape,
                       jax.ShapeDtypeStruct((n, c), jnp.bfloat16)],
        )(*args, w_next)
    return pl.pallas_call(
        functools.partial(_gru_body, c),
        grid=(n // bn,),
        in_specs=in_specs,
        out_specs=out_specs,
        out_shape=out_shape,
    )(*args)


def _sc_gather_sum(mw, idx2, deg, nc, ns, dpw0):
    npad, cw = mw.shape
    c = 2 * cw
    rows_per_step = idx2.shape[1]
    sub = rows_per_step // deg
    dpw1 = npad // ns - dpw0
    steps0 = dpw0 // sub
    steps1 = dpw1 // sub
    steps_max = max(steps0, steps1)
    dpw_max = max(dpw0, dpw1)
    core0_total = ns * dpw0
    wgroups = cw // _LANES
    mesh = plsc.VectorSubcoreMesh(core_axis_name="c", subcore_axis_name="s")

    @functools.partial(
        pl.kernel,
        out_type=jax.ShapeDtypeStruct((npad, c), jnp.float32),
        mesh=mesh,
        compiler_params=pltpu.CompilerParams(use_tc_tiling_on_sc=False),
        scratch_types=[
            pltpu.VMEM((steps_max, rows_per_step), jnp.int32),
            pltpu.VMEM((4, rows_per_step, cw), jnp.int32),
            pltpu.VMEM((dpw_max, c), jnp.float32),
            pltpu.SemaphoreType.DMA,
            pltpu.SemaphoreType.DMA,
            pltpu.SemaphoreType.DMA,
            pltpu.SemaphoreType.DMA,
        ],
    )
    def k(m_hbm, idx_hbm, out_hbm, idx_v, rows_v, out_v,
          sem0, sem1, sem2, sem3):
        sid = lax.axis_index("s")
        cid = lax.axis_index("c")
        on0 = cid == 0
        dst0 = jnp.where(on0, sid * dpw0, core0_total + sid * dpw1)
        mysteps = jnp.where(on0, steps0, steps1)
        row0 = dst0 // sub

        if steps0 > 0:
            @pl.when(on0)
            def _():
                pltpu.sync_copy(idx_hbm.at[pl.ds(row0, steps0)],
                                idx_v.at[pl.ds(0, steps0)])

        if steps1 > 0:
            @pl.when(jnp.logical_not(on0))
            def _():
                pltpu.sync_copy(idx_hbm.at[pl.ds(row0, steps1)],
                                idx_v.at[pl.ds(0, steps1)])

        sems = (sem0, sem1, sem2, sem3)
        nbuf = len(sems)
        himask = jnp.full((_LANES,), -65536, jnp.int32)

        def start(g, b):
            pltpu.async_copy(m_hbm.at[idx_v.at[g]], rows_v.at[b], sems[b])

        def wait(g, b):
            pltpu.make_async_copy(m_hbm.at[idx_v.at[g]], rows_v.at[b],
                                  sems[b]).wait()

        @pl.when(mysteps >= nbuf)
        def _():
            for b in range(nbuf):
                start(b, b)

        def body(i, carry):
            for b in range(nbuf):
                g = i * nbuf + b
                wait(g, b)
                unroll = 8
                for d in range(sub):
                    def nbody(jo, acc):
                        r0 = d * deg + jo * unroll
                        for u in range(unroll):
                            for v in range(wgroups):
                                w = rows_v[b, r0 + u,
                                           pl.ds(v * _LANES, _LANES)]
                                lo = lax.bitcast_convert_type(
                                    w << 16, jnp.float32)
                                hi = lax.bitcast_convert_type(
                                    w & himask, jnp.float32)
                                acc = (acc[:2 * v]
                                       + (acc[2 * v] + lo,
                                          acc[2 * v + 1] + hi)
                                       + acc[2 * v + 2:])
                        return acc
                    acc = lax.fori_loop(
                        0, deg // unroll, nbody,
                        tuple(jnp.zeros((_LANES,), jnp.float32)
                              for _ in range(2 * wgroups)))
                    row_out = g * sub + d
                    for v in range(wgroups):
                        out_v[row_out,
                              pl.ds(v * 32, _LANES)] = acc[2 * v]
                        out_v[row_out,
                              pl.ds(v * 32 + _LANES, _LANES)] = acc[2 * v + 1]

                @pl.when(g + nbuf < mysteps)
                def _():
                    start(g + nbuf, b)
            return carry

        lax.fori_loop(0, mysteps // nbuf, body, 0)

        if dpw0 > 0:
            @pl.when(on0)
            def _():
                pltpu.sync_copy(out_v.at[pl.ds(0, dpw0)],
                                out_hbm.at[pl.ds(dst0, dpw0)])

        if dpw1 > 0:
            @pl.when(jnp.logical_not(on0))
            def _():
                pltpu.sync_copy(out_v.at[pl.ds(0, dpw1)],
                                out_hbm.at[pl.ds(dst0, dpw1)])

    return k(mw, idx2)


def _perm(c):
    p = []
    for v in range(c // 32):
        p.extend(v * 32 + 2 * k for k in range(16))
        p.extend(v * 32 + 2 * k + 1 for k in range(16))
    return np.asarray(p, np.int32)


def kernel(x, edge_index, weight, W_ih, W_hh, b_ih, b_hh):
    n, c = x.shape
    deg = edge_index.shape[1]
    num_layers = weight.shape[0]
    info = plsc.get_sparse_core_info()
    nc, ns = info.num_cores, info.num_subcores
    nw = nc * ns

    rows_per_step = 128
    sub = rows_per_step // deg
    per_w = sub * nw
    steps = -(-n // per_w)
    steps = -(-steps // 4) * 4
    npad = steps * per_w

    xp = jnp.concatenate(
        [x, jnp.zeros((npad - n, c), jnp.float32)], axis=0)
    ei = jnp.concatenate(
        [edge_index, jnp.zeros((npad - n, deg), jnp.int32)], axis=0)
    idx2 = ei.reshape(npad // sub, rows_per_step)
    dpw0 = (npad // ns) * 95 // 100 // 16 * 16

    wih_t = W_ih.T[_perm(c)]
    whh_t = W_hh.T
    bih = b_ih.reshape(1, -1)
    bhh = b_hh.reshape(1, -1)

    bn = 256
    m = _tc_matmul(xp, weight[0], bn, jnp.bfloat16)
    for i in range(num_layers):
        mw = lax.bitcast_convert_type(
            m.reshape(npad, c // 2, 2), jnp.int32)
        agg = _sc_gather_sum(mw, idx2, deg, nc, ns, dpw0)
        if i + 1 < num_layers:
            xp, m = _tc_gru(agg, xp, wih_t, whh_t, bih, bhh, bn,
                            w_next=weight[i + 1])
        else:
            xp = _tc_gru(agg, xp, wih_t, whh_t, bih, bhh, bn)
    return xp[:n]


# speedup vs baseline: 1.2933x; 1.0754x over previous
import functools

import numpy as np

import jax
import jax.numpy as jnp
from jax import lax
from jax.experimental import pallas as pl
from jax.experimental.pallas import tpu as pltpu
from jax.experimental.pallas import tpu_sc as plsc

_LANES = 16


def _matmul_body(x_ref, w_ref, o_ref):
    o_ref[...] = jnp.dot(x_ref[...], w_ref[...],
                         preferred_element_type=jnp.float32
                         ).astype(o_ref.dtype)


def _tc_matmul(x, w, bn, out_dtype):
    n, k = x.shape
    kk, m = w.shape
    return pl.pallas_call(
        _matmul_body,
        grid=(n // bn,),
        in_specs=[
            pl.BlockSpec((bn, k), lambda i: (i, 0)),
            pl.BlockSpec((kk, m), lambda i: (0, 0)),
        ],
        out_specs=pl.BlockSpec((bn, m), lambda i: (i, 0)),
        out_shape=jax.ShapeDtypeStruct((n, m), out_dtype),
    )(x, w)


def _gru_body(c, agg_ref, h_ref, wih_ref, whh_ref, bih_ref, bhh_ref, *o_refs):
    h = h_ref[...]
    gi = jnp.dot(agg_ref[...], wih_ref[...],
                 preferred_element_type=jnp.float32) + bih_ref[...]
    gh = jnp.dot(h, whh_ref[...],
                 preferred_element_type=jnp.float32) + bhh_ref[...]
    r = jax.nn.sigmoid(gi[:, :c] + gh[:, :c])
    z = jax.nn.sigmoid(gi[:, c:2 * c] + gh[:, c:2 * c])
    nn = jnp.tanh(gi[:, 2 * c:] + r * gh[:, 2 * c:])
    hn = (1.0 - z) * nn + z * h
    o_refs[0][...] = hn
    if len(o_refs) > 1:
        wn_ref = o_refs[2]
        o_refs[1][...] = jnp.dot(
            hn, wn_ref[...],
            preferred_element_type=jnp.float32).astype(jnp.bfloat16)


def _tc_gru(agg, h, wih_t, whh_t, bih, bhh, bn, w_next=None):
    n, c = h.shape
    g3 = wih_t.shape[1]
    in_specs = [
        pl.BlockSpec((bn, c), lambda i: (i, 0)),
        pl.BlockSpec((bn, c), lambda i: (i, 0)),
        pl.BlockSpec((c, g3), lambda i: (0, 0)),
        pl.BlockSpec((c, g3), lambda i: (0, 0)),
        pl.BlockSpec((1, g3), lambda i: (0, 0)),
        pl.BlockSpec((1, g3), lambda i: (0, 0)),
    ]
    out_specs = pl.BlockSpec((bn, c), lambda i: (i, 0))
    out_shape = jax.ShapeDtypeStruct((n, c), jnp.float32)
    args = [agg, h, wih_t, whh_t, bih, bhh]
    if w_next is not None:

        def body(c_, agg_r, h_r, wih_r, whh_r, bih_r, bhh_r, wn_r, o1, o2):
            _gru_body(c_, agg_r, h_r, wih_r, whh_r, bih_r, bhh_r, o1, o2, wn_r)

        return pl.pallas_call(
            functools.partial(body, c),
            grid=(n // bn,),
            in_specs=in_specs + [pl.BlockSpec((c, c), lambda i: (0, 0))],
            out_specs=[out_specs,
                       pl.BlockSpec((bn, c), lambda i: (i, 0))],
            out_shape=[out_shape,
                       jax.ShapeDtypeStruct((n, c), jnp.bfloat16)],
        )(*args, w_next)
    return pl.pallas_call(
        functools.partial(_gru_body, c),
        grid=(n // bn,),
        in_specs=in_specs,
        out_specs=out_specs,
        out_shape=out_shape,
    )(*args)


def _sc_gather_sum(mw, idx2, deg, nc, ns, dpw0):
    npad, cw = mw.shape
    c = 2 * cw
    rows_per_step = idx2.shape[1]
    sub = rows_per_step // deg
    dpw1 = npad // ns - dpw0
    steps0 = dpw0 // sub
    steps1 = dpw1 // sub
    steps_max = max(steps0, steps1)
    dpw_max = max(dpw0, dpw1)
    core0_total = ns * dpw0
    wgroups = cw // _LANES
    mesh = plsc.VectorSubcoreMesh(core_axis_name="c", subcore_axis_name="s")

    @functools.partial(
        pl.kernel,
        out_type=jax.ShapeDtypeStruct((npad, c), jnp.float32),
        mesh=mesh,
        compiler_params=pltpu.CompilerParams(use_tc_tiling_on_sc=False),
        scratch_types=[
            pltpu.VMEM((steps_max, rows_per_step), jnp.int32),
            pltpu.VMEM((4, rows_per_step, cw), jnp.int32),
            pltpu.VMEM((dpw_max, c), jnp.float32),
            pltpu.SemaphoreType.DMA,
            pltpu.SemaphoreType.DMA,
            pltpu.SemaphoreType.DMA,
            pltpu.SemaphoreType.DMA,
        ],
    )
    def k(m_hbm, idx_hbm, out_hbm, idx_v, rows_v, out_v,
          sem0, sem1, sem2, sem3):
        sid = lax.axis_index("s")
        cid = lax.axis_index("c")
        on0 = cid == 0
        dst0 = jnp.where(on0, sid * dpw0, core0_total + sid * dpw1)
        mysteps = jnp.where(on0, steps0, steps1)
        row0 = dst0 // sub

        if steps0 > 0:
            @pl.when(on0)
            def _():
                pltpu.sync_copy(idx_hbm.at[pl.ds(row0, steps0)],
                                idx_v.at[pl.ds(0, steps0)])

        if steps1 > 0:
            @pl.when(jnp.logical_not(on0))
            def _():
                pltpu.sync_copy(idx_hbm.at[pl.ds(row0, steps1)],
                                idx_v.at[pl.ds(0, steps1)])

        sems = (sem0, sem1, sem2, sem3)
        nbuf = len(sems)
        himask = jnp.full((_LANES,), -65536, jnp.int32)

        def start(g, b):
            pltpu.async_copy(m_hbm.at[idx_v.at[g]], rows_v.at[b], sems[b])

        def wait(g, b):
            pltpu.make_async_copy(m_hbm.at[idx_v.at[g]], rows_v.at[b],
                                  sems[b]).wait()

        @pl.when(mysteps >= nbuf)
        def _():
            for b in range(nbuf):
                start(b, b)

        def body(i, carry):
            for b in range(nbuf):
                g = i * nbuf + b
                wait(g, b)
                unroll = 8
                for d in range(sub):
                    def nbody(jo, acc):
                        r0 = d * deg + jo * unroll
                        for u in range(unroll):
                            for v in range(wgroups):
                                w = rows_v[b, r0 + u,
                                           pl.ds(v * _LANES, _LANES)]
                                lo = lax.bitcast_convert_type(
                                    w << 16, jnp.float32)
                                hi = lax.bitcast_convert_type(
                                    w & himask, jnp.float32)
                                acc = (acc[:2 * v]
                                       + (acc[2 * v] + lo,
                                          acc[2 * v + 1] + hi)
                                       + acc[2 * v + 2:])
                        return acc
                    acc = lax.fori_loop(
                        0, deg // unroll, nbody,
                        tuple(jnp.zeros((_LANES,), jnp.float32)
                              for _ in range(2 * wgroups)))
                    row_out = g * sub + d
                    for v in range(wgroups):
                        out_v[row_out,
                              pl.ds(v * 32, _LANES)] = acc[2 * v]
                        out_v[row_out,
                              pl.ds(v * 32 + _LANES, _LANES)] = acc[2 * v + 1]

                @pl.when(g + nbuf < mysteps)
                def _():
                    start(g + nbuf, b)
            return carry

        lax.fori_loop(0, mysteps // nbuf, body, 0)

        if dpw0 > 0:
            @pl.when(on0)
            def _():
                pltpu.sync_copy(out_v.at[pl.ds(0, dpw0)],
                                out_hbm.at[pl.ds(dst0, dpw0)])

        if dpw1 > 0:
            @pl.when(jnp.logical_not(on0))
            def _():
                pltpu.sync_copy(out_v.at[pl.ds(0, dpw1)],
                                out_hbm.at[pl.ds(dst0, dpw1)])

    return k(mw, idx2)


def _perm(c):
    p = []
    for v in range(c // 32):
        p.extend(v * 32 + 2 * k for k in range(16))
        p.extend(v * 32 + 2 * k + 1 for k in range(16))
    return np.asarray(p, np.int32)


def kernel(x, edge_index, weight, W_ih, W_hh, b_ih, b_hh):
    n, c = x.shape
    deg = edge_index.shape[1]
    num_layers = weight.shape[0]
    info = plsc.get_sparse_core_info()
    nc, ns = info.num_cores, info.num_subcores
    nw = nc * ns

    rows_per_step = 128
    sub = rows_per_step // deg
    per_w = sub * nw
    steps = -(-n // per_w)
    steps = -(-steps // 4) * 4
    npad = steps * per_w

    xp = jnp.concatenate(
        [x, jnp.zeros((npad - n, c), jnp.float32)], axis=0)
    ei = jnp.concatenate(
        [edge_index, jnp.zeros((npad - n, deg), jnp.int32)], axis=0)
    idx2 = ei.reshape(npad // sub, rows_per_step)
    dpw0 = (npad // ns) * 95 // 100 // 16 * 16

    wih_t = W_ih.T[_perm(c)]
    whh_t = W_hh.T
    bih = b_ih.reshape(1, -1)
    bhh = b_hh.reshape(1, -1)

    bn = 512
    m = _tc_matmul(xp, weight[0], bn, jnp.bfloat16)
    for i in range(num_layers):
        mw = lax.bitcast_convert_type(
            m.reshape(npad, c // 2, 2), jnp.int32)
        agg = _sc_gather_sum(mw, idx2, deg, nc, ns, dpw0)
        if i + 1 < num_layers:
            xp, m = _tc_gru(agg, xp, wih_t, whh_t, bih, bhh, bn,
                            w_next=weight[i + 1])
        else:
            xp = _tc_gru(agg, xp, wih_t, whh_t, bih, bhh, bn)
    return xp[:n]


# speedup vs baseline: 1.3086x; 1.0118x over previous
import functools

import numpy as np

import jax
import jax.numpy as jnp
from jax import lax
from jax.experimental import pallas as pl
from jax.experimental.pallas import tpu as pltpu
from jax.experimental.pallas import tpu_sc as plsc

_LANES = 16


def _matmul_body(x_ref, w_ref, o_ref):
    o_ref[...] = jnp.dot(x_ref[...], w_ref[...],
                         preferred_element_type=jnp.float32
                         ).astype(o_ref.dtype)


def _tc_matmul(x, w, bn, out_dtype):
    n, k = x.shape
    kk, m = w.shape
    return pl.pallas_call(
        _matmul_body,
        grid=(n // bn,),
        in_specs=[
            pl.BlockSpec((bn, k), lambda i: (i, 0)),
            pl.BlockSpec((kk, m), lambda i: (0, 0)),
        ],
        out_specs=pl.BlockSpec((bn, m), lambda i: (i, 0)),
        out_shape=jax.ShapeDtypeStruct((n, m), out_dtype),
    )(x, w)


def _gru_body(c, agg_ref, h_ref, wih_ref, whh_ref, bih_ref, bhh_ref, *o_refs):
    h = h_ref[...]
    gi = jnp.dot(agg_ref[...], wih_ref[...],
                 preferred_element_type=jnp.float32) + bih_ref[...]
    gh = jnp.dot(h, whh_ref[...],
                 preferred_element_type=jnp.float32) + bhh_ref[...]
    r = jax.nn.sigmoid(gi[:, :c] + gh[:, :c])
    z = jax.nn.sigmoid(gi[:, c:2 * c] + gh[:, c:2 * c])
    nn = jnp.tanh(gi[:, 2 * c:] + r * gh[:, 2 * c:])
    hn = (1.0 - z) * nn + z * h
    o_refs[0][...] = hn
    if len(o_refs) > 1:
        wn_ref = o_refs[2]
        o_refs[1][...] = jnp.dot(
            hn, wn_ref[...],
            preferred_element_type=jnp.float32).astype(jnp.bfloat16)


def _tc_gru(agg, h, wih_t, whh_t, bih, bhh, bn, w_next=None):
    n, c = h.shape
    g3 = wih_t.shape[1]
    in_specs = [
        pl.BlockSpec((bn, c), lambda i: (i, 0)),
        pl.BlockSpec((bn, c), lambda i: (i, 0)),
        pl.BlockSpec((c, g3), lambda i: (0, 0)),
        pl.BlockSpec((c, g3), lambda i: (0, 0)),
        pl.BlockSpec((1, g3), lambda i: (0, 0)),
        pl.BlockSpec((1, g3), lambda i: (0, 0)),
    ]
    out_specs = pl.BlockSpec((bn, c), lambda i: (i, 0))
    out_shape = jax.ShapeDtypeStruct((n, c), jnp.float32)
    args = [agg, h, wih_t, whh_t, bih, bhh]
    if w_next is not None:

        def body(c_, agg_r, h_r, wih_r, whh_r, bih_r, bhh_r, wn_r, o1, o2):
            _gru_body(c_, agg_r, h_r, wih_r, whh_r, bih_r, bhh_r, o1, o2, wn_r)

        return pl.pallas_call(
            functools.partial(body, c),
            grid=(n // bn,),
            in_specs=in_specs + [pl.BlockSpec((c, c), lambda i: (0, 0))],
            out_specs=[out_specs,
                       pl.BlockSpec((bn, c), lambda i: (i, 0))],
            out_shape=[out_shape,
                       jax.ShapeDtypeStruct((n, c), jnp.bfloat16)],
        )(*args, w_next)
    return pl.pallas_call(
        functools.partial(_gru_body, c),
        grid=(n // bn,),
        in_specs=in_specs,
        out_specs=out_specs,
        out_shape=out_shape,
    )(*args)


def _sc_gather_sum(mw, idx2, deg, nc, ns, dpw0):
    npad, cw = mw.shape
    c = 2 * cw
    rows_per_step = idx2.shape[1]
    sub = rows_per_step // deg
    dpw1 = npad // ns - dpw0
    steps0 = dpw0 // sub
    steps1 = dpw1 // sub
    steps_max = max(steps0, steps1)
    dpw_max = max(dpw0, dpw1)
    core0_total = ns * dpw0
    wgroups = cw // _LANES
    mesh = plsc.VectorSubcoreMesh(core_axis_name="c", subcore_axis_name="s")

    @functools.partial(
        pl.kernel,
        out_type=jax.ShapeDtypeStruct((npad, c), jnp.float32),
        mesh=mesh,
        compiler_params=pltpu.CompilerParams(use_tc_tiling_on_sc=False),
        scratch_types=[
            pltpu.VMEM((steps_max, rows_per_step), jnp.int32),
            pltpu.VMEM((4, rows_per_step, cw), jnp.int32),
            pltpu.VMEM((dpw_max, c), jnp.float32),
            pltpu.SemaphoreType.DMA,
            pltpu.SemaphoreType.DMA,
            pltpu.SemaphoreType.DMA,
            pltpu.SemaphoreType.DMA,
        ],
    )
    def k(m_hbm, idx_hbm, out_hbm, idx_v, rows_v, out_v,
          sem0, sem1, sem2, sem3):
        sid = lax.axis_index("s")
        cid = lax.axis_index("c")
        on0 = cid == 0
        dst0 = jnp.where(on0, sid * dpw0, core0_total + sid * dpw1)
        mysteps = jnp.where(on0, steps0, steps1)
        row0 = dst0 // sub

        if steps0 > 0:
            @pl.when(on0)
            def _():
                pltpu.sync_copy(idx_hbm.at[pl.ds(row0, steps0)],
                                idx_v.at[pl.ds(0, steps0)])

        if steps1 > 0:
            @pl.when(jnp.logical_not(on0))
            def _():
                pltpu.sync_copy(idx_hbm.at[pl.ds(row0, steps1)],
                                idx_v.at[pl.ds(0, steps1)])

        sems = (sem0, sem1, sem2, sem3)
        nbuf = len(sems)
        himask = jnp.full((_LANES,), -65536, jnp.int32)

        def start(g, b):
            pltpu.async_copy(m_hbm.at[idx_v.at[g]], rows_v.at[b], sems[b])

        def wait(g, b):
            pltpu.make_async_copy(m_hbm.at[idx_v.at[g]], rows_v.at[b],
                                  sems[b]).wait()

        @pl.when(mysteps >= nbuf)
        def _():
            for b in range(nbuf):
                start(b, b)

        def body(i, carry):
            for b in range(nbuf):
                g = i * nbuf + b
                wait(g, b)
                unroll = 8
                for d in range(sub):
                    def nbody(jo, acc):
                        r0 = d * deg + jo * unroll
                        for u in range(unroll):
                            for v in range(wgroups):
                                w = rows_v[b, r0 + u,
                                           pl.ds(v * _LANES, _LANES)]
                                lo = lax.bitcast_convert_type(
                                    w << 16, jnp.float32)
                                hi = lax.bitcast_convert_type(
                                    w & himask, jnp.float32)
                                acc = (acc[:2 * v]
                                       + (acc[2 * v] + lo,
                                          acc[2 * v + 1] + hi)
                                       + acc[2 * v + 2:])
                        return acc
                    acc = lax.fori_loop(
                        0, deg // unroll, nbody,
                        tuple(jnp.zeros((_LANES,), jnp.float32)
                              for _ in range(2 * wgroups)))
                    row_out = g * sub + d
                    for v in range(wgroups):
                        out_v[row_out,
                              pl.ds(v * 32, _LANES)] = acc[2 * v]
                        out_v[row_out,
                              pl.ds(v * 32 + _LANES, _LANES)] = acc[2 * v + 1]

                @pl.when(g + nbuf < mysteps)
                def _():
                    start(g + nbuf, b)
            return carry

        lax.fori_loop(0, mysteps // nbuf, body, 0)

        if dpw0 > 0:
            @pl.when(on0)
            def _():
                pltpu.sync_copy(out_v.at[pl.ds(0, dpw0)],
                                out_hbm.at[pl.ds(dst0, dpw0)])

        if dpw1 > 0:
            @pl.when(jnp.logical_not(on0))
            def _():
                pltpu.sync_copy(out_v.at[pl.ds(0, dpw1)],
                                out_hbm.at[pl.ds(dst0, dpw1)])

    return k(mw, idx2)


def _perm(c):
    p = []
    for v in range(c // 32):
        p.extend(v * 32 + 2 * k for k in range(16))
        p.extend(v * 32 + 2 * k + 1 for k in range(16))
    return np.asarray(p, np.int32)


def kernel(x, edge_index, weight, W_ih, W_hh, b_ih, b_hh):
    n, c = x.shape
    deg = edge_index.shape[1]
    num_layers = weight.shape[0]
    info = plsc.get_sparse_core_info()
    nc, ns = info.num_cores, info.num_subcores
    nw = nc * ns

    rows_per_step = 128
    sub = rows_per_step // deg
    per_w = sub * nw
    steps = -(-n // per_w)
    steps = -(-steps // 4) * 4
    npad = steps * per_w

    xp = jnp.concatenate(
        [x, jnp.zeros((npad - n, c), jnp.float32)], axis=0)
    ei = jnp.concatenate(
        [edge_index, jnp.zeros((npad - n, deg), jnp.int32)], axis=0)
    idx2 = ei.reshape(npad // sub, rows_per_step)
    dpw0 = (npad // ns) * 95 // 100 // 16 * 16

    wih_t = W_ih.T[_perm(c)]
    whh_t = W_hh.T
    bih = b_ih.reshape(1, -1)
    bhh = b_hh.reshape(1, -1)

    bn = 1024
    m = _tc_matmul(xp, weight[0], bn, jnp.bfloat16)
    for i in range(num_layers):
        mw = lax.bitcast_convert_type(
            m.reshape(npad, c // 2, 2), jnp.int32)
        agg = _sc_gather_sum(mw, idx2, deg, nc, ns, dpw0)
        if i + 1 < num_layers:
            xp, m = _tc_gru(agg, xp, wih_t, whh_t, bih, bhh, bn,
                            w_next=weight[i + 1])
        else:
            xp = _tc_gru(agg, xp, wih_t, whh_t, bih, bhh, bn)
    return xp[:n]


# speedup vs baseline: 1.3112x; 1.0020x over previous
import functools

import numpy as np

import jax
import jax.numpy as jnp
from jax import lax
from jax.experimental import pallas as pl
from jax.experimental.pallas import tpu as pltpu
from jax.experimental.pallas import tpu_sc as plsc

_LANES = 16


def _matmul_body(x_ref, w_ref, o_ref):
    o_ref[...] = jnp.dot(x_ref[...], w_ref[...],
                         preferred_element_type=jnp.float32
                         ).astype(o_ref.dtype)


def _tc_matmul(x, w, bn, out_dtype):
    n, k = x.shape
    kk, m = w.shape
    return pl.pallas_call(
        _matmul_body,
        grid=(n // bn,),
        in_specs=[
            pl.BlockSpec((bn, k), lambda i: (i, 0)),
            pl.BlockSpec((kk, m), lambda i: (0, 0)),
        ],
        out_specs=pl.BlockSpec((bn, m), lambda i: (i, 0)),
        out_shape=jax.ShapeDtypeStruct((n, m), out_dtype),
    )(x, w)


def _gru_body(c, agg_ref, h_ref, wih_ref, whh_ref, bih_ref, bhh_ref, *o_refs):
    h = h_ref[...]
    gi = jnp.dot(agg_ref[...], wih_ref[...],
                 preferred_element_type=jnp.float32) + bih_ref[...]
    gh = jnp.dot(h, whh_ref[...],
                 preferred_element_type=jnp.float32) + bhh_ref[...]
    r = jax.nn.sigmoid(gi[:, :c] + gh[:, :c])
    z = jax.nn.sigmoid(gi[:, c:2 * c] + gh[:, c:2 * c])
    nn = jnp.tanh(gi[:, 2 * c:] + r * gh[:, 2 * c:])
    hn = (1.0 - z) * nn + z * h
    o_refs[0][...] = hn
    if len(o_refs) > 1:
        wn_ref = o_refs[2]
        o_refs[1][...] = jnp.dot(
            hn, wn_ref[...],
            preferred_element_type=jnp.float32).astype(jnp.bfloat16)


def _tc_gru(agg, h, wih_t, whh_t, bih, bhh, bn, w_next=None):
    n, c = h.shape
    g3 = wih_t.shape[1]
    in_specs = [
        pl.BlockSpec((bn, c), lambda i: (i, 0)),
        pl.BlockSpec((bn, c), lambda i: (i, 0)),
        pl.BlockSpec((c, g3), lambda i: (0, 0)),
        pl.BlockSpec((c, g3), lambda i: (0, 0)),
        pl.BlockSpec((1, g3), lambda i: (0, 0)),
        pl.BlockSpec((1, g3), lambda i: (0, 0)),
    ]
    out_specs = pl.BlockSpec((bn, c), lambda i: (i, 0))
    out_shape = jax.ShapeDtypeStruct((n, c), jnp.float32)
    args = [agg, h, wih_t, whh_t, bih, bhh]
    if w_next is not None:

        def body(c_, agg_r, h_r, wih_r, whh_r, bih_r, bhh_r, wn_r, o1, o2):
            _gru_body(c_, agg_r, h_r, wih_r, whh_r, bih_r, bhh_r, o1, o2, wn_r)

        return pl.pallas_call(
            functools.partial(body, c),
            grid=(n // bn,),
            in_specs=in_specs + [pl.BlockSpec((c, c), lambda i: (0, 0))],
            out_specs=[out_specs,
                       pl.BlockSpec((bn, c), lambda i: (i, 0))],
            out_shape=[out_shape,
                       jax.ShapeDtypeStruct((n, c), jnp.bfloat16)],
        )(*args, w_next)
    return pl.pallas_call(
        functools.partial(_gru_body, c),
        grid=(n // bn,),
        in_specs=in_specs,
        out_specs=out_specs,
        out_shape=out_shape,
    )(*args)


def _sc_gather_sum(mw, idx2, deg, nc, ns, dpw0):
    npad, cw = mw.shape
    c = 2 * cw
    rows_per_step = idx2.shape[1]
    sub = rows_per_step // deg
    dpw1 = npad // ns - dpw0
    steps0 = dpw0 // sub
    steps1 = dpw1 // sub
    steps_max = max(steps0, steps1)
    dpw_max = max(dpw0, dpw1)
    core0_total = ns * dpw0
    wgroups = cw // _LANES
    mesh = plsc.VectorSubcoreMesh(core_axis_name="c", subcore_axis_name="s")

    @functools.partial(
        pl.kernel,
        out_type=jax.ShapeDtypeStruct((npad, c), jnp.float32),
        mesh=mesh,
        compiler_params=pltpu.CompilerParams(use_tc_tiling_on_sc=False),
        scratch_types=[
            pltpu.VMEM((steps_max, rows_per_step), jnp.int32),
            pltpu.VMEM((4, rows_per_step, cw), jnp.int32),
            pltpu.VMEM((dpw_max, c), jnp.float32),
            pltpu.SemaphoreType.DMA,
            pltpu.SemaphoreType.DMA,
            pltpu.SemaphoreType.DMA,
            pltpu.SemaphoreType.DMA,
        ],
    )
    def k(m_hbm, idx_hbm, out_hbm, idx_v, rows_v, out_v,
          sem0, sem1, sem2, sem3):
        sid = lax.axis_index("s")
        cid = lax.axis_index("c")
        on0 = cid == 0
        dst0 = jnp.where(on0, sid * dpw0, core0_total + sid * dpw1)
        mysteps = jnp.where(on0, steps0, steps1)
        row0 = dst0 // sub

        if steps0 > 0:
            @pl.when(on0)
            def _():
                pltpu.sync_copy(idx_hbm.at[pl.ds(row0, steps0)],
                                idx_v.at[pl.ds(0, steps0)])

        if steps1 > 0:
            @pl.when(jnp.logical_not(on0))
            def _():
                pltpu.sync_copy(idx_hbm.at[pl.ds(row0, steps1)],
                                idx_v.at[pl.ds(0, steps1)])

        sems = (sem0, sem1, sem2, sem3)
        nbuf = len(sems)
        himask = jnp.full((_LANES,), -65536, jnp.int32)

        def start(g, b):
            pltpu.async_copy(m_hbm.at[idx_v.at[g]], rows_v.at[b], sems[b])

        def wait(g, b):
            pltpu.make_async_copy(m_hbm.at[idx_v.at[g]], rows_v.at[b],
                                  sems[b]).wait()

        @pl.when(mysteps >= nbuf)
        def _():
            for b in range(nbuf):
                start(b, b)

        def body(i, carry):
            for b in range(nbuf):
                g = i * nbuf + b
                wait(g, b)
                unroll = 8
                for d in range(sub):
                    def nbody(jo, acc):
                        r0 = d * deg + jo * unroll
                        for u in range(unroll):
                            for v in range(wgroups):
                                w = rows_v[b, r0 + u,
                                           pl.ds(v * _LANES, _LANES)]
                                lo = lax.bitcast_convert_type(
                                    w << 16, jnp.float32)
                                hi = lax.bitcast_convert_type(
                                    w & himask, jnp.float32)
                                acc = (acc[:2 * v]
                                       + (acc[2 * v] + lo,
                                          acc[2 * v + 1] + hi)
                                       + acc[2 * v + 2:])
                        return acc
                    acc = lax.fori_loop(
                        0, deg // unroll, nbody,
                        tuple(jnp.zeros((_LANES,), jnp.float32)
                              for _ in range(2 * wgroups)))
                    row_out = g * sub + d
                    for v in range(wgroups):
                        out_v[row_out,
                              pl.ds(v * 32, _LANES)] = acc[2 * v]
                        out_v[row_out,
                              pl.ds(v * 32 + _LANES, _LANES)] = acc[2 * v + 1]

                @pl.when(g + nbuf < mysteps)
                def _():
                    start(g + nbuf, b)
            return carry

        lax.fori_loop(0, mysteps // nbuf, body, 0)

        if dpw0 > 0:
            @pl.when(on0)
            def _():
                pltpu.sync_copy(out_v.at[pl.ds(0, dpw0)],
                                out_hbm.at[pl.ds(dst0, dpw0)])

        if dpw1 > 0:
            @pl.when(jnp.logical_not(on0))
            def _():
                pltpu.sync_copy(out_v.at[pl.ds(0, dpw1)],
                                out_hbm.at[pl.ds(dst0, dpw1)])

    return k(mw, idx2)


def _perm(c):
    p = []
    for v in range(c // 32):
        p.extend(v * 32 + 2 * k for k in range(16))
        p.extend(v * 32 + 2 * k + 1 for k in range(16))
    return np.asarray(p, np.int32)


def kernel(x, edge_index, weight, W_ih, W_hh, b_ih, b_hh):
    n, c = x.shape
    deg = edge_index.shape[1]
    num_layers = weight.shape[0]
    info = plsc.get_sparse_core_info()
    nc, ns = info.num_cores, info.num_subcores
    nw = nc * ns

    rows_per_step = 128
    sub = rows_per_step // deg
    per_w = sub * nw
    steps = -(-n // per_w)
    steps = -(-steps // 4) * 4
    npad = steps * per_w

    xp = jnp.concatenate(
        [x, jnp.zeros((npad - n, c), jnp.float32)], axis=0)
    ei = jnp.concatenate(
        [edge_index, jnp.zeros((npad - n, deg), jnp.int32)], axis=0)
    idx2 = ei.reshape(npad // sub, rows_per_step)
    dpw0 = (npad // ns) * 95 // 100 // 16 * 16

    wih_t = W_ih.T[_perm(c)]
    whh_t = W_hh.T
    bih = b_ih.reshape(1, -1)
    bhh = b_hh.reshape(1, -1)

    bn = 2048
    m = _tc_matmul(xp, weight[0], bn, jnp.bfloat16)
    for i in range(num_layers):
        mw = lax.bitcast_convert_type(
            m.reshape(npad, c // 2, 2), jnp.int32)
        agg = _sc_gather_sum(mw, idx2, deg, nc, ns, dpw0)
        if i + 1 < num_layers:
            xp, m = _tc_gru(agg, xp, wih_t, whh_t, bih, bhh, bn,
                            w_next=weight[i + 1])
        else:
            xp = _tc_gru(agg, xp, wih_t, whh_t, bih, bhh, bn)
    return xp[:n]


# speedup vs baseline: 1.3462x; 1.0267x over previous
import functools

import numpy as np

import jax
import jax.numpy as jnp
from jax import lax
from jax.experimental import pallas as pl
from jax.experimental.pallas import tpu as pltpu
from jax.experimental.pallas import tpu_sc as plsc

_LANES = 16


def _matmul_body(x_ref, w_ref, o_ref):
    o_ref[...] = jnp.dot(x_ref[...], w_ref[...],
                         preferred_element_type=jnp.float32
                         ).astype(o_ref.dtype)


def _tc_matmul(x, w, bn, out_dtype):
    n, k = x.shape
    kk, m = w.shape
    return pl.pallas_call(
        _matmul_body,
        grid=(n // bn,),
        in_specs=[
            pl.BlockSpec((bn, k), lambda i: (i, 0)),
            pl.BlockSpec((kk, m), lambda i: (0, 0)),
        ],
        out_specs=pl.BlockSpec((bn, m), lambda i: (i, 0)),
        out_shape=jax.ShapeDtypeStruct((n, m), out_dtype),
    )(x, w)


def _gru_body(c, agg_ref, h_ref, wih_ref, whh_ref, bih_ref, bhh_ref, *o_refs):
    h = h_ref[...]
    gi = jnp.dot(agg_ref[...], wih_ref[...],
                 preferred_element_type=jnp.float32) + bih_ref[...]
    gh = jnp.dot(h, whh_ref[...],
                 preferred_element_type=jnp.float32) + bhh_ref[...]
    r = jax.nn.sigmoid(gi[:, :c] + gh[:, :c])
    z = jax.nn.sigmoid(gi[:, c:2 * c] + gh[:, c:2 * c])
    nn = jnp.tanh(gi[:, 2 * c:] + r * gh[:, 2 * c:])
    hn = (1.0 - z) * nn + z * h
    o_refs[0][...] = hn
    if len(o_refs) > 1:
        wn_ref = o_refs[2]
        o_refs[1][...] = jnp.dot(
            hn, wn_ref[...],
            preferred_element_type=jnp.float32).astype(jnp.bfloat16)


def _tc_gru(agg, h, wih_t, whh_t, bih, bhh, bn, w_next=None):
    n, c = h.shape
    g3 = wih_t.shape[1]
    in_specs = [
        pl.BlockSpec((bn, c), lambda i: (i, 0)),
        pl.BlockSpec((bn, c), lambda i: (i, 0)),
        pl.BlockSpec((c, g3), lambda i: (0, 0)),
        pl.BlockSpec((c, g3), lambda i: (0, 0)),
        pl.BlockSpec((1, g3), lambda i: (0, 0)),
        pl.BlockSpec((1, g3), lambda i: (0, 0)),
    ]
    out_specs = pl.BlockSpec((bn, c), lambda i: (i, 0))
    out_shape = jax.ShapeDtypeStruct((n, c), jnp.float32)
    args = [agg, h, wih_t, whh_t, bih, bhh]
    if w_next is not None:

        def body(c_, agg_r, h_r, wih_r, whh_r, bih_r, bhh_r, wn_r, o1, o2):
            _gru_body(c_, agg_r, h_r, wih_r, whh_r, bih_r, bhh_r, o1, o2, wn_r)

        return pl.pallas_call(
            functools.partial(body, c),
            grid=(n // bn,),
            in_specs=in_specs + [pl.BlockSpec((c, c), lambda i: (0, 0))],
            out_specs=[out_specs,
                       pl.BlockSpec((bn, c), lambda i: (i, 0))],
            out_shape=[out_shape,
                       jax.ShapeDtypeStruct((n, c), jnp.bfloat16)],
        )(*args, w_next)
    return pl.pallas_call(
        functools.partial(_gru_body, c),
        grid=(n // bn,),
        in_specs=in_specs,
        out_specs=out_specs,
        out_shape=out_shape,
    )(*args)


def _sc_gather_sum(mw, idx2, deg, nc, ns, dpw0):
    npad, cw = mw.shape
    c = 2 * cw
    rows_per_step = idx2.shape[1]
    sub = rows_per_step // deg
    dpw1 = npad // ns - dpw0
    steps0 = dpw0 // sub
    steps1 = dpw1 // sub
    steps_max = max(steps0, steps1)
    dpw_max = max(dpw0, dpw1)
    core0_total = ns * dpw0
    wgroups = cw // _LANES
    mesh = plsc.VectorSubcoreMesh(core_axis_name="c", subcore_axis_name="s")

    @functools.partial(
        pl.kernel,
        out_type=jax.ShapeDtypeStruct((npad, c), jnp.float32),
        mesh=mesh,
        compiler_params=pltpu.CompilerParams(use_tc_tiling_on_sc=False),
        scratch_types=[
            pltpu.VMEM((steps_max, rows_per_step), jnp.int32),
            pltpu.VMEM((4, rows_per_step, cw), jnp.int32),
            pltpu.VMEM((dpw_max, c), jnp.float32),
            pltpu.SemaphoreType.DMA,
            pltpu.SemaphoreType.DMA,
            pltpu.SemaphoreType.DMA,
            pltpu.SemaphoreType.DMA,
        ],
    )
    def k(m_hbm, idx_hbm, out_hbm, idx_v, rows_v, out_v,
          sem0, sem1, sem2, sem3):
        sid = lax.axis_index("s")
        cid = lax.axis_index("c")
        on0 = cid == 0
        dst0 = jnp.where(on0, sid * dpw0, core0_total + sid * dpw1)
        mysteps = jnp.where(on0, steps0, steps1)
        row0 = dst0 // sub

        if steps0 > 0:
            @pl.when(on0)
            def _():
                pltpu.sync_copy(idx_hbm.at[pl.ds(row0, steps0)],
                                idx_v.at[pl.ds(0, steps0)])

        if steps1 > 0:
            @pl.when(jnp.logical_not(on0))
            def _():
                pltpu.sync_copy(idx_hbm.at[pl.ds(row0, steps1)],
                                idx_v.at[pl.ds(0, steps1)])

        sems = (sem0, sem1, sem2, sem3)
        nbuf = len(sems)
        himask = jnp.full((_LANES,), -65536, jnp.int32)

        def start(g, b):
            pltpu.async_copy(m_hbm.at[idx_v.at[g]], rows_v.at[b], sems[b])

        def wait(g, b):
            pltpu.make_async_copy(m_hbm.at[idx_v.at[g]], rows_v.at[b],
                                  sems[b]).wait()

        @pl.when(mysteps >= nbuf)
        def _():
            for b in range(nbuf):
                start(b, b)

        def body(i, carry):
            for b in range(nbuf):
                g = i * nbuf + b
                wait(g, b)
                unroll = 8
                for d in range(sub):
                    def nbody(jo, acc):
                        r0 = d * deg + jo * unroll
                        for u in range(unroll):
                            for v in range(wgroups):
                                w = rows_v[b, r0 + u,
                                           pl.ds(v * _LANES, _LANES)]
                                lo = lax.bitcast_convert_type(
                                    w << 16, jnp.float32)
                                hi = lax.bitcast_convert_type(
                                    w & himask, jnp.float32)
                                acc = (acc[:2 * v]
                                       + (acc[2 * v] + lo,
                                          acc[2 * v + 1] + hi)
                                       + acc[2 * v + 2:])
                        return acc
                    acc = lax.fori_loop(
                        0, deg // unroll, nbody,
                        tuple(jnp.zeros((_LANES,), jnp.float32)
                              for _ in range(2 * wgroups)))
                    row_out = g * sub + d
                    for v in range(wgroups):
                        out_v[row_out,
                              pl.ds(v * 32, _LANES)] = acc[2 * v]
                        out_v[row_out,
                              pl.ds(v * 32 + _LANES, _LANES)] = acc[2 * v + 1]

                @pl.when(g + nbuf < mysteps)
                def _():
                    start(g + nbuf, b)
            return carry

        lax.fori_loop(0, mysteps // nbuf, body, 0)

        if dpw0 > 0:
            @pl.when(on0)
            def _():
                pltpu.sync_copy(out_v.at[pl.ds(0, dpw0)],
                                out_hbm.at[pl.ds(dst0, dpw0)])

        if dpw1 > 0:
            @pl.when(jnp.logical_not(on0))
            def _():
                pltpu.sync_copy(out_v.at[pl.ds(0, dpw1)],
                                out_hbm.at[pl.ds(dst0, dpw1)])

    return k(mw, idx2)


def _perm(c):
    p = []
    for v in range(c // 32):
        p.extend(v * 32 + 2 * k for k in range(16))
        p.extend(v * 32 + 2 * k + 1 for k in range(16))
    return np.asarray(p, np.int32)


def kernel(x, edge_index, weight, W_ih, W_hh, b_ih, b_hh):
    n, c = x.shape
    deg = edge_index.shape[1]
    num_layers = weight.shape[0]
    info = plsc.get_sparse_core_info()
    nc, ns = info.num_cores, info.num_subcores
    nw = nc * ns

    rows_per_step = 128
    sub = rows_per_step // deg
    per_w = sub * nw
    steps = -(-n // per_w)
    steps = -(-steps // 4) * 4
    npad = steps * per_w

    xp = jnp.concatenate(
        [x, jnp.zeros((npad - n, c), jnp.float32)], axis=0)
    ei = jnp.concatenate(
        [edge_index, jnp.zeros((npad - n, deg), jnp.int32)], axis=0)
    idx2 = ei.reshape(npad // sub, rows_per_step)
    dpw0 = (npad // ns) * 95 // 100 // 16 * 16

    wih_t = W_ih.T[_perm(c)]
    whh_t = W_hh.T
    bih = b_ih.reshape(1, -1)
    bhh = b_hh.reshape(1, -1)

    bn = next(b for b in (2048, 1024, 512, 256, 128) if npad % b == 0)
    m = _tc_matmul(xp, weight[0], bn, jnp.bfloat16)
    for i in range(num_layers):
        mw = lax.bitcast_convert_type(
            m.reshape(npad, c // 2, 2), jnp.int32)
        agg = _sc_gather_sum(mw, idx2, deg, nc, ns, dpw0)
        if i + 1 < num_layers:
            xp, m = _tc_gru(agg, xp, wih_t, whh_t, bih, bhh, bn,
                            w_next=weight[i + 1])
        else:
            xp = _tc_gru(agg, xp, wih_t, whh_t, bih, bhh, bn)
    return xp[:n]
